# baseline (speedup 1.0000x reference)
import functools

import jax
import jax.numpy as jnp
from jax import lax
from jax.experimental import pallas as pl
from jax.experimental.pallas import tpu as pltpu
from jax.experimental.pallas import tpu_sc as plsc

VOCAB = 1000000
EMBED = 64
BATCH = 4096

NC = 2
NS = 16
LANES = 16
NW = NC * NS
B_PER_W = BATCH // NW


def _sc_gather(word_u, word_v, in_embed, in_bias_flat, out_embed,
               out_bias_flat):
    mesh = plsc.VectorSubcoreMesh(core_axis_name="c", subcore_axis_name="s")

    @functools.partial(
        pl.kernel,
        mesh=mesh,
        compiler_params=pltpu.CompilerParams(use_tc_tiling_on_sc=False),
        out_type=(
            jax.ShapeDtypeStruct((BATCH, EMBED), jnp.float32),
            jax.ShapeDtypeStruct((BATCH, EMBED), jnp.float32),
            jax.ShapeDtypeStruct((BATCH,), jnp.float32),
        ),
        scratch_types=[
            pltpu.VMEM((B_PER_W,), jnp.int32),
            pltpu.VMEM((B_PER_W,), jnp.int32),
            pltpu.VMEM((B_PER_W, EMBED), jnp.float32),
            pltpu.VMEM((B_PER_W, EMBED), jnp.float32),
            pltpu.VMEM((B_PER_W,), jnp.float32),
            pltpu.VMEM((B_PER_W,), jnp.float32),
            pltpu.VMEM((B_PER_W,), jnp.float32),
            pltpu.SemaphoreType.DMA,
            pltpu.SemaphoreType.DMA,
            pltpu.SemaphoreType.DMA,
            pltpu.SemaphoreType.DMA,
        ],
    )
    def k(word_u_hbm, word_v_hbm, in_embed_hbm, in_bias_hbm, out_embed_hbm,
          out_bias_hbm, u_rows_hbm, v_rows_hbm, b_hbm, idx_u, idx_v,
          rows_u, rows_v, bias_u, bias_v, b_loc, sem0, sem1, sem2, sem3):
        wid = lax.axis_index("s") * NC + lax.axis_index("c")
        base = wid * B_PER_W

        pltpu.sync_copy(word_u_hbm.at[pl.ds(base, B_PER_W)], idx_u)
        pltpu.sync_copy(word_v_hbm.at[pl.ds(base, B_PER_W)], idx_v)

        c0 = pltpu.async_copy(in_embed_hbm.at[idx_u], rows_u, sem0)
        c1 = pltpu.async_copy(out_embed_hbm.at[idx_v], rows_v, sem1)
        c2 = pltpu.async_copy(in_bias_hbm.at[idx_u], bias_u, sem2)
        c3 = pltpu.async_copy(out_bias_hbm.at[idx_v], bias_v, sem3)
        c2.wait()
        c3.wait()

        for g in range(B_PER_W // LANES):
            sl = pl.ds(g * LANES, LANES)
            b_loc[sl] = bias_u[sl] + bias_v[sl]
        pltpu.sync_copy(b_loc, b_hbm.at[pl.ds(base, B_PER_W)])

        c0.wait()
        c1.wait()
        pltpu.sync_copy(rows_u, u_rows_hbm.at[pl.ds(base, B_PER_W), :])
        pltpu.sync_copy(rows_v, v_rows_hbm.at[pl.ds(base, B_PER_W), :])

    return k(word_u, word_v, in_embed, in_bias_flat, out_embed,
             out_bias_flat)


def _tc_dot(u_rows, v_rows):
    def body(u_ref, v_ref, s_ref):
        s_ref[...] = jnp.sum(u_ref[...] * v_ref[...], axis=1, keepdims=True)

    return pl.pallas_call(
        body,
        out_shape=jax.ShapeDtypeStruct((BATCH, 1), jnp.float32),
    )(u_rows, v_rows)


def _tc_outer_add(b_col, s_row):
    BM = 256

    def body(b_ref, s_ref, o_ref):
        o_ref[...] = b_ref[...] + s_ref[...]

    return pl.pallas_call(
        body,
        grid=(BATCH // BM,),
        in_specs=[
            pl.BlockSpec((BM, 1), lambda i: (i, 0)),
            pl.BlockSpec((1, BATCH), lambda i: (0, 0)),
        ],
        out_specs=pl.BlockSpec((BM, BATCH), lambda i: (i, 0)),
        out_shape=jax.ShapeDtypeStruct((BATCH, BATCH), jnp.float32),
    )(b_col, s_row)


def kernel(word_u, word_v, in_embed, in_bias, out_embed, out_bias):
    word_u = word_u.astype(jnp.int32)
    word_v = word_v.astype(jnp.int32)
    u_rows, v_rows, b = _sc_gather(word_u, word_v, in_embed,
                                   in_bias.reshape(VOCAB), out_embed,
                                   out_bias.reshape(VOCAB))
    s_col = _tc_dot(u_rows, v_rows)
    return _tc_outer_add(b.reshape(BATCH, 1), s_col.reshape(1, BATCH))


# speedup vs baseline: 1.3990x; 1.3990x over previous
import functools

import jax
import jax.numpy as jnp
from jax import lax
from jax.experimental import pallas as pl
from jax.experimental.pallas import tpu as pltpu
from jax.experimental.pallas import tpu_sc as plsc

VOCAB = 1000000
EMBED = 64
BATCH = 4096

NC = 2
NS = 16
LANES = 16
NW = NC * NS
B_PER_W = BATCH // NW


def _sc_gather(word_u, word_v, in_embed, in_bias_flat, out_embed,
               out_bias_flat):
    mesh = plsc.VectorSubcoreMesh(core_axis_name="c", subcore_axis_name="s")

    @functools.partial(
        pl.kernel,
        mesh=mesh,
        out_type=(
            jax.ShapeDtypeStruct((BATCH, EMBED), jnp.float32),
            jax.ShapeDtypeStruct((BATCH, EMBED), jnp.float32),
            jax.ShapeDtypeStruct((BATCH,), jnp.float32),
        ),
        scratch_types=[
            pltpu.VMEM((B_PER_W,), jnp.int32),
            pltpu.VMEM((B_PER_W,), jnp.int32),
            pltpu.VMEM((B_PER_W, EMBED), jnp.float32),
            pltpu.VMEM((B_PER_W, EMBED), jnp.float32),
            pltpu.VMEM((B_PER_W,), jnp.float32),
            pltpu.VMEM((B_PER_W,), jnp.float32),
            pltpu.VMEM((B_PER_W,), jnp.float32),
            pltpu.SemaphoreType.DMA,
            pltpu.SemaphoreType.DMA,
            pltpu.SemaphoreType.DMA,
            pltpu.SemaphoreType.DMA,
        ],
    )
    def k(word_u_hbm, word_v_hbm, in_embed_hbm, in_bias_hbm, out_embed_hbm,
          out_bias_hbm, u_rows_hbm, v_rows_hbm, b_hbm, idx_u, idx_v,
          rows_u, rows_v, bias_u, bias_v, b_loc, sem_u, sem_v, sem2, sem3):
        wid = lax.axis_index("s") * NC + lax.axis_index("c")
        base = wid * B_PER_W

        pltpu.sync_copy(word_u_hbm.at[pl.ds(base, B_PER_W)], idx_u)
        pltpu.sync_copy(word_v_hbm.at[pl.ds(base, B_PER_W)], idx_v)

        c2 = pltpu.async_copy(in_bias_hbm.at[idx_u], bias_u, sem2)
        c3 = pltpu.async_copy(out_bias_hbm.at[idx_v], bias_v, sem3)

        copies = []
        for g in range(B_PER_W // LANES):
            iu = idx_u[pl.ds(g * LANES, LANES)]
            iv = idx_v[pl.ds(g * LANES, LANES)]
            for i in range(LANES):
                r = g * LANES + i
                copies.append(pltpu.async_copy(
                    in_embed_hbm.at[pl.ds(iu[i], 1), :],
                    rows_u.at[pl.ds(r, 1), :], sem_u))
                copies.append(pltpu.async_copy(
                    out_embed_hbm.at[pl.ds(iv[i], 1), :],
                    rows_v.at[pl.ds(r, 1), :], sem_v))
        for c in copies:
            c.wait()

        c2.wait()
        c3.wait()
        for g in range(B_PER_W // LANES):
            sl = pl.ds(g * LANES, LANES)
            b_loc[sl] = bias_u[sl] + bias_v[sl]
        pltpu.sync_copy(b_loc, b_hbm.at[pl.ds(base, B_PER_W)])

        pltpu.sync_copy(rows_u, u_rows_hbm.at[pl.ds(base, B_PER_W), :])
        pltpu.sync_copy(rows_v, v_rows_hbm.at[pl.ds(base, B_PER_W), :])

    return k(word_u, word_v, in_embed, in_bias_flat, out_embed,
             out_bias_flat)


def _tc_dot(u_rows, v_rows):
    def body(u_ref, v_ref, s_ref):
        s_ref[...] = jnp.sum(u_ref[...] * v_ref[...], axis=1, keepdims=True)

    return pl.pallas_call(
        body,
        out_shape=jax.ShapeDtypeStruct((BATCH, 1), jnp.float32),
    )(u_rows, v_rows)


def _tc_outer_add(b_col, s_row):
    BM = 256

    def body(b_ref, s_ref, o_ref):
        o_ref[...] = b_ref[...] + s_ref[...]

    return pl.pallas_call(
        body,
        grid=(BATCH // BM,),
        in_specs=[
            pl.BlockSpec((BM, 1), lambda i: (i, 0)),
            pl.BlockSpec((1, BATCH), lambda i: (0, 0)),
        ],
        out_specs=pl.BlockSpec((BM, BATCH), lambda i: (i, 0)),
        out_shape=jax.ShapeDtypeStruct((BATCH, BATCH), jnp.float32),
    )(b_col, s_row)


def kernel(word_u, word_v, in_embed, in_bias, out_embed, out_bias):
    word_u = word_u.astype(jnp.int32)
    word_v = word_v.astype(jnp.int32)
    u_rows, v_rows, b = _sc_gather(word_u, word_v, in_embed,
                                   in_bias.reshape(VOCAB), out_embed,
                                   out_bias.reshape(VOCAB))
    s_col = _tc_dot(u_rows, v_rows)
    return _tc_outer_add(b.reshape(BATCH, 1), s_col.reshape(1, BATCH))
